# baseline (speedup 1.0000x reference)
import functools

import jax
import jax.numpy as jnp
from jax import lax
from jax.experimental import pallas as pl
from jax.experimental.pallas import tpu as pltpu
from jax.experimental.pallas import tpu_sc as plsc

B = 16384
D = 32
NC = 2
NS = 16
L = 16
NW = NC * NS
BPW = B // NW
CHUNKS = BPW // L


def _sc_body(uid_hbm, iid_hbm, nid_hbm, umem_hbm, imem_hbm,
             score_hbm, nscore_hbm,
             uidx_v, iidx_v, nidx_v, urows_v, irows_v, nrows_v,
             score_v, nscore_v, sem):
    wid = lax.axis_index("s") * NC + lax.axis_index("c")
    base = wid * BPW

    pltpu.sync_copy(uid_hbm.at[pl.ds(base, BPW)], uidx_v)
    pltpu.sync_copy(iid_hbm.at[pl.ds(base, BPW)], iidx_v)
    pltpu.sync_copy(nid_hbm.at[pl.ds(base, BPW)], nidx_v)

    cu = pltpu.async_copy(umem_hbm.at[uidx_v], urows_v, sem)
    ci = pltpu.async_copy(imem_hbm.at[iidx_v], irows_v, sem)
    cn = pltpu.async_copy(imem_hbm.at[nidx_v], nrows_v, sem)
    cu.wait()
    ci.wait()
    cn.wait()

    iota = lax.iota(jnp.int32, L)

    def chunk_body(k, carry):
        rows = k * L + iota
        acc_s = jnp.zeros((L,), jnp.float32)
        acc_n = jnp.zeros((L,), jnp.float32)
        for c in range(D):
            col = jnp.full((L,), c, jnp.int32)
            uv = plsc.load_gather(urows_v, [rows, col])
            iv = plsc.load_gather(irows_v, [rows, col])
            nv = plsc.load_gather(nrows_v, [rows, col])
            acc_s = acc_s + uv * iv
            acc_n = acc_n + uv * nv
        score_v[pl.ds(k * L, L)] = acc_s
        nscore_v[pl.ds(k * L, L)] = acc_n
        return carry

    lax.fori_loop(0, CHUNKS, chunk_body, 0)

    pltpu.sync_copy(score_v, score_hbm.at[pl.ds(base, BPW)])
    pltpu.sync_copy(nscore_v, nscore_hbm.at[pl.ds(base, BPW)])


def kernel(user_id, item_id, neg_item_id, user_memory, item_memory):
    mesh = plsc.VectorSubcoreMesh(core_axis_name="c", subcore_axis_name="s")
    run = functools.partial(
        pl.kernel,
        mesh=mesh,
        out_type=(jax.ShapeDtypeStruct((B,), jnp.float32),
                  jax.ShapeDtypeStruct((B,), jnp.float32)),
        scratch_types=[
            pltpu.VMEM((BPW,), jnp.int32),
            pltpu.VMEM((BPW,), jnp.int32),
            pltpu.VMEM((BPW,), jnp.int32),
            pltpu.VMEM((BPW, D), jnp.float32),
            pltpu.VMEM((BPW, D), jnp.float32),
            pltpu.VMEM((BPW, D), jnp.float32),
            pltpu.VMEM((BPW,), jnp.float32),
            pltpu.VMEM((BPW,), jnp.float32),
            pltpu.SemaphoreType.DMA,
        ],
        compiler_params=pltpu.CompilerParams(needs_layout_passes=False,
                                             use_tc_tiling_on_sc=False),
    )(_sc_body)
    return run(user_id.astype(jnp.int32), item_id.astype(jnp.int32),
               neg_item_id.astype(jnp.int32), user_memory, item_memory)
